# baseline (speedup 1.0000x reference)
import functools
import jax
import jax.numpy as jnp
from jax.experimental import pallas as pl
from jax.experimental.pallas import tpu as pltpu

_B, _T, _DIN, _H, _C = 1024, 85, 20, 1024, 27
_OBJ = _T * 36
_UNROLL = 6


def _base_kernel(o_ref, oW1_ref, ob1_ref, oW2_ref, ob2_ref, eW1_ref,
                 eb1_ref, base_ref):
    h = jnp.dot(o_ref[...], oW1_ref[...],
                preferred_element_type=jnp.float32) + ob1_ref[...]
    h = jnp.maximum(h, 0.0)
    obj = jnp.dot(h, oW2_ref[...],
                  preferred_element_type=jnp.float32) + ob2_ref[...]
    objpad = jnp.concatenate(
        [jnp.zeros((obj.shape[0], _DIN), jnp.float32), obj], axis=1)
    base_ref[...] = jnp.dot(objpad, eW1_ref[...],
                            preferred_element_type=jnp.float32) + eb1_ref[...]


def _loop_kernel(xT_ref, base_ref, eW1_ref, w2_ref, eb2_ref, prev_in_ref,
                 out_ref, prev_out_ref, prev_ref, *, first_chunk):
    c = pl.program_id(1)

    @pl.when(c == 0)
    def _():
        if first_chunk:
            prev_ref[...] = xT_ref[0][:, 0:1].astype(jnp.bfloat16)
        else:
            prev_ref[...] = prev_in_ref[...]

    prev = prev_ref[...]
    w1 = eW1_ref[0:_DIN, :].astype(jnp.bfloat16)
    w2 = w2_ref[...].astype(jnp.bfloat16)
    base = base_ref[...]
    for s in range(_UNROLL):
        xt = xT_ref[s].astype(jnp.bfloat16)
        cur = jnp.concatenate([prev, xt[:, 1:]], axis=1)
        h = jnp.dot(cur, w1, preferred_element_type=jnp.float32) + base
        h = jnp.maximum(h, 0.0).astype(jnp.bfloat16)
        ew = jnp.dot(h, w2,
                     preferred_element_type=jnp.float32) + eb2_ref[...]
        out_ref[s] = ew
        prev = ew[:, 0:1].astype(jnp.bfloat16)
    prev_ref[...] = prev
    prev_out_ref[...] = prev


def kernel(o, x, oW1, ob1, oW2, ob2, eW1, eb1, eW2, eb2):
    MB = 256
    base = pl.pallas_call(
        _base_kernel,
        grid=(_B // MB,),
        in_specs=[
            pl.BlockSpec((MB, _OBJ), lambda i: (i, 0)),
            pl.BlockSpec((_OBJ, _H), lambda i: (0, 0)),
            pl.BlockSpec((1, _H), lambda i: (0, 0)),
            pl.BlockSpec((_H, _H), lambda i: (0, 0)),
            pl.BlockSpec((1, _H), lambda i: (0, 0)),
            pl.BlockSpec((_DIN + _H, _H), lambda i: (0, 0)),
            pl.BlockSpec((1, _H), lambda i: (0, 0)),
        ],
        out_specs=pl.BlockSpec((MB, _H), lambda i: (i, 0)),
        out_shape=jax.ShapeDtypeStruct((_B, _H), jnp.float32),
        compiler_params=pltpu.CompilerParams(
            dimension_semantics=("parallel",),
            vmem_limit_bytes=56 * 1024 * 1024,
        ),
        name="walk_base",
    )(o, oW1, ob1.reshape(1, -1), oW2, ob2.reshape(1, -1), eW1,
      eb1.reshape(1, -1))

    xT = jnp.swapaxes(x, 0, 1)
    BB = 512
    eb2r = eb2.reshape(1, -1)

    def _chunk(prev_in, first, toff, nsteps):
        return pl.pallas_call(
            functools.partial(_loop_kernel, first_chunk=first),
            grid=(_B // BB, nsteps // _UNROLL),
            in_specs=[
                pl.BlockSpec((_UNROLL, BB, _DIN),
                             lambda b, t, o=toff: (t + o, b, 0)),
                pl.BlockSpec((BB, _H), lambda b, t: (b, 0)),
                pl.BlockSpec((_DIN + _H, _H), lambda b, t: (0, 0)),
                pl.BlockSpec((_H, _C), lambda b, t: (0, 0)),
                pl.BlockSpec((1, _C), lambda b, t: (0, 0)),
                pl.BlockSpec((BB, 1), lambda b, t: (b, 0)),
            ],
            out_specs=[
                pl.BlockSpec((_UNROLL, BB, _C), lambda b, t: (t, b, 0)),
                pl.BlockSpec((BB, 1), lambda b, t: (b, 0)),
            ],
            out_shape=[
                jax.ShapeDtypeStruct((nsteps, _B, _C), jnp.float32),
                jax.ShapeDtypeStruct((_B, 1), jnp.bfloat16),
            ],
            scratch_shapes=[pltpu.VMEM((BB, 1), jnp.bfloat16)],
            compiler_params=pltpu.CompilerParams(
                dimension_semantics=("parallel", "arbitrary"),
            ),
            name="walk_loop",
        )(xT, base, eW1, eW2, eb2r, prev_in)

    half = (_T - 1) // 2
    zero_prev = jnp.zeros((_B, 1), jnp.bfloat16)
    outA, prevA = _chunk(zero_prev, True, 0, half)
    outB, _ = _chunk(prevA, False, half // _UNROLL, half)
    return jnp.concatenate(
        [jnp.swapaxes(outA, 0, 1), jnp.swapaxes(outB, 0, 1)], axis=1)
